# baseline (speedup 1.0000x reference)
import functools

import jax
import jax.numpy as jnp
from jax import lax
from jax.experimental import pallas as pl
from jax.experimental.pallas import tpu as pltpu
from jax.experimental.pallas import tpu_sc as plsc

N = 10000
E = 320000
D_IN = 128
D_HID = 128
N_CLS = 40

NP = 10240
HW = 64
CP = 48
NC = 2
NS = 16
CHUNK = 128
NCHUNKS = E // CHUNK
BASE = NCHUNKS // 32
XTRA = NCHUNKS % 32
ROWS_PER_TILE = NP // NS
NBUF = 6

_SC_PARAMS = pltpu.CompilerParams(use_tc_tiling_on_sc=False)


def _sc_mesh():
    return plsc.VectorSubcoreMesh(
        core_axis_name="c", subcore_axis_name="s", num_cores=NC, num_subcores=NS
    )


def _tile_chunks(gid):
    start = gid * BASE + lax.min(gid, XTRA)
    return start, gid < XTRA


def _stage_indices(e3, which, idx_v, gid):
    start, extra = _tile_chunks(gid)
    pltpu.sync_copy(e3.at[which, pl.ds(start, BASE)], idx_v.at[pl.ds(0, BASE)])

    @pl.when(extra)
    def _():
        pltpu.sync_copy(e3.at[which, pl.ds(start + BASE, 1)],
                        idx_v.at[pl.ds(BASE, 1)])


def _make_degree_kernel():
    @functools.partial(
        pl.kernel,
        out_type=(
            jax.ShapeDtypeStruct((NC, NP), jnp.float32),
            jax.ShapeDtypeStruct((NC, NP), jnp.float32),
        ),
        mesh=_sc_mesh(),
        compiler_params=_SC_PARAMS,
        scratch_types=[
            pltpu.VMEM((BASE + 1, CHUNK), jnp.int32),
            pltpu.VMEM((BASE + 1, CHUNK), jnp.int32),
            pltpu.VMEM((CHUNK,), jnp.float32),
            pltpu.VMEM_SHARED((NP,), jnp.float32),
            pltpu.VMEM_SHARED((NP,), jnp.float32),
            pltpu.SemaphoreType.DMA,
            pltpu.SemaphoreType.DMA,
        ],
    )
    def deg_kernel(e3_hbm, ones_hbm, zeros_hbm, dog_hbm, dig_hbm,
                   src_v, dst_v, ones_v, acc_o, acc_i, sem_o, sem_i):
        c = lax.axis_index("c")
        s = lax.axis_index("s")
        gid = c * NS + s
        _stage_indices(e3_hbm, 0, src_v, gid)
        _stage_indices(e3_hbm, 1, dst_v, gid)
        pltpu.sync_copy(ones_hbm, ones_v)
        sl = pl.ds(s * ROWS_PER_TILE, ROWS_PER_TILE)
        pltpu.sync_copy(zeros_hbm, acc_o.at[sl])
        pltpu.sync_copy(zeros_hbm, acc_i.at[sl])
        plsc.subcore_barrier()
        _, extra = _tile_chunks(gid)

        def body(j, carry):
            pltpu.async_copy(ones_v, acc_o.at[src_v.at[j]], sem_o, add=True)
            pltpu.async_copy(ones_v, acc_i.at[dst_v.at[j]], sem_i, add=True)
            return carry

        lax.fori_loop(0, BASE, body, 0)

        @pl.when(extra)
        def _():
            pltpu.async_copy(ones_v, acc_o.at[src_v.at[BASE]], sem_o, add=True)
            pltpu.async_copy(ones_v, acc_i.at[dst_v.at[BASE]], sem_i, add=True)

        def drain(j, carry):
            pltpu.make_async_copy(ones_v, acc_o.at[src_v.at[j]], sem_o).wait()
            pltpu.make_async_copy(ones_v, acc_i.at[dst_v.at[j]], sem_i).wait()
            return carry

        lax.fori_loop(0, BASE, drain, 0)

        @pl.when(extra)
        def _():
            pltpu.make_async_copy(ones_v, acc_o.at[src_v.at[BASE]],
                                  sem_o).wait()
            pltpu.make_async_copy(ones_v, acc_i.at[dst_v.at[BASE]],
                                  sem_i).wait()

        plsc.subcore_barrier()
        pltpu.sync_copy(acc_o.at[sl], dog_hbm.at[c, sl])
        pltpu.sync_copy(acc_i.at[sl], dig_hbm.at[c, sl])

    return deg_kernel


def _make_agg_kernel(nh, d):
    @functools.partial(
        pl.kernel,
        out_type=jax.ShapeDtypeStruct((NC, NP, 128), jnp.float32),
        mesh=_sc_mesh(),
        compiler_params=_SC_PARAMS,
        scratch_types=[
            pltpu.VMEM((BASE + 1, CHUNK), jnp.int32),
            pltpu.VMEM((BASE + 1, CHUNK), jnp.int32),
            [pltpu.VMEM((CHUNK, d), jnp.float32) for _ in range(NBUF)],
            pltpu.VMEM_SHARED((NP, d), jnp.float32),
            [pltpu.SemaphoreType.DMA for _ in range(NBUF)],
            [pltpu.SemaphoreType.DMA for _ in range(NBUF)],
        ],
    )
    def agg_kernel(*refs):
        feats = refs[:nh]
        e3_hbm, zeros_hbm, out_hbm = refs[nh:nh + 3]
        src_v, dst_v, rows, acc, gsem, ssem = refs[nh + 3:]
        c = lax.axis_index("c")
        s = lax.axis_index("s")
        gid = c * NS + s
        _stage_indices(e3_hbm, 0, src_v, gid)
        _stage_indices(e3_hbm, 1, dst_v, gid)
        _, extra = _tile_chunks(gid)
        sl = pl.ds(s * ROWS_PER_TILE, ROWS_PER_TILE)

        for h in range(nh):
            feat = feats[h]
            pltpu.sync_copy(zeros_hbm, acc.at[sl])
            plsc.subcore_barrier()

            @pl.when(extra)
            def _():
                pltpu.async_copy(feat.at[src_v.at[BASE]], rows[0], gsem[0])
                pltpu.make_async_copy(feat.at[src_v.at[BASE]], rows[0],
                                      gsem[0]).wait()
                pltpu.sync_copy(rows[0], acc.at[dst_v.at[BASE]], add=True)

            for b in range(NBUF):
                pltpu.async_copy(feat.at[src_v.at[b]], rows[b], gsem[b])

            def body(jj, carry):
                j = jj * NBUF
                for b in range(NBUF):
                    pltpu.make_async_copy(feat.at[src_v.at[j + b]], rows[b],
                                          gsem[b]).wait()
                    pltpu.async_copy(rows[b], acc.at[dst_v.at[j + b]],
                                     ssem[b], add=True)

                    @pl.when(j + b + NBUF < BASE)
                    def _():
                        pltpu.make_async_copy(rows[b],
                                              acc.at[dst_v.at[j + b]],
                                              ssem[b]).wait()
                        pltpu.async_copy(feat.at[src_v.at[j + b + NBUF]],
                                         rows[b], gsem[b])
                return carry

            lax.fori_loop(0, BASE // NBUF, body, 0)
            for b in range(NBUF):
                pltpu.make_async_copy(rows[b],
                                      acc.at[dst_v.at[BASE - NBUF + b]],
                                      ssem[b]).wait()
            plsc.subcore_barrier()
            pltpu.sync_copy(acc.at[sl],
                            out_hbm.at[c, sl, pl.ds(h * d, d)])

    return agg_kernel


def _tc_mm_body(x_ref, w1_ref, xw_ref):
    xw_ref[...] = jnp.dot(x_ref[...], w1_ref[...],
                          preferred_element_type=jnp.float32)


def _tc_a_body(xw_ref, dog_ref, dig_ref,
               xwa_ref, xwb_ref, normo_ref, normi_ref):
    do = dog_ref[0, :] + dog_ref[1, :]
    di = dig_ref[0, :] + dig_ref[1, :]
    no = jnp.where(do > 0, lax.rsqrt(do), 0.0)
    ni = jnp.where(di > 0, lax.rsqrt(di), 0.0)
    normo_ref[0, :] = no
    normi_ref[0, :] = ni
    xw = xw_ref[...] * no[:N, None]
    xwa_ref[...] = xw[:, :HW]
    xwb_ref[...] = xw[:, HW:]


def _tc_b_body(p_ref, normi_ref, normo_ref, b1_ref, w2_ref, hw_ref):
    ni = normi_ref[0, :N]
    no = normo_ref[0, :N]
    a = p_ref[0, :N, :] + p_ref[1, :N, :]
    h = jnp.maximum(a * ni[:, None] + b1_ref[0, :][None, :], 0.0)
    hw = jnp.dot(h, w2_ref[...], preferred_element_type=jnp.float32)
    hw_ref[...] = hw * no[:, None]


def _tc_c_body(p_ref, normi_ref, b2_ref, out_ref):
    ni = normi_ref[0, :N]
    z = (p_ref[0, :N, :CP] + p_ref[1, :N, :CP]) * ni[:, None] \
        + b2_ref[0, :][None, :]
    col = lax.broadcasted_iota(jnp.int32, z.shape, 1)
    valid = col < N_CLS
    zm = jnp.where(valid, z, -jnp.inf)
    m = jnp.max(zm, axis=1, keepdims=True)
    e = jnp.where(valid, jnp.exp(z - m), 0.0)
    ssum = jnp.sum(e, axis=1, keepdims=True)
    out_ref[...] = (z - m - jnp.log(ssum))[:, :N_CLS]


def kernel(inputs, edge_index, W1, b1, W2, b2):
    x = inputs.astype(jnp.float32)
    e3 = edge_index.astype(jnp.int32).reshape(2, NCHUNKS, CHUNK)

    w2_p = jnp.pad(W2.astype(jnp.float32), ((0, 0), (0, CP - N_CLS)))
    b1_r = b1.astype(jnp.float32).reshape(1, D_HID)
    b2_r = jnp.pad(b2.astype(jnp.float32), (0, CP - N_CLS)).reshape(1, CP)

    ones_c = jnp.ones((CHUNK,), jnp.float32)
    zeros_t = jnp.zeros((ROWS_PER_TILE,), jnp.float32)
    zeros_th = jnp.zeros((ROWS_PER_TILE, HW), jnp.float32)
    zeros_tc = jnp.zeros((ROWS_PER_TILE, CP), jnp.float32)

    dog, dig = _make_degree_kernel()(e3, ones_c, zeros_t)

    xw = pl.pallas_call(
        _tc_mm_body,
        out_shape=jax.ShapeDtypeStruct((N, D_HID), jnp.float32),
    )(x, W1.astype(jnp.float32))

    xwa, xwb, normo, normi = pl.pallas_call(
        _tc_a_body,
        out_shape=[
            jax.ShapeDtypeStruct((N, HW), jnp.float32),
            jax.ShapeDtypeStruct((N, HW), jnp.float32),
            jax.ShapeDtypeStruct((1, NP), jnp.float32),
            jax.ShapeDtypeStruct((1, NP), jnp.float32),
        ],
    )(xw, dog, dig)

    agg1 = _make_agg_kernel(2, HW)(xwa, xwb, e3, zeros_th)

    hw2 = pl.pallas_call(
        _tc_b_body,
        out_shape=jax.ShapeDtypeStruct((N, CP), jnp.float32),
    )(agg1, normi, normo, b1_r, w2_p)

    agg2 = _make_agg_kernel(1, CP)(hw2, e3, zeros_tc)

    out = pl.pallas_call(
        _tc_c_body,
        out_shape=jax.ShapeDtypeStruct((N, N_CLS), jnp.float32),
    )(agg2, normi, b2_r)

    return out


# speedup vs baseline: 1.0079x; 1.0079x over previous
import functools

import jax
import jax.numpy as jnp
from jax import lax
from jax.experimental import pallas as pl
from jax.experimental.pallas import tpu as pltpu
from jax.experimental.pallas import tpu_sc as plsc

N = 10000
E = 320000
D_IN = 128
D_HID = 128
N_CLS = 40

NP = 10240
HW = 64
CP = 48
NC = 2
NS = 16
CHUNK = 128
NCHUNKS = E // CHUNK
BASE = NCHUNKS // 32
XTRA = NCHUNKS % 32
ROWS_PER_TILE = NP // NS
NBUF = 6

_SC_PARAMS = pltpu.CompilerParams(use_tc_tiling_on_sc=False)


def _sc_mesh():
    return plsc.VectorSubcoreMesh(
        core_axis_name="c", subcore_axis_name="s", num_cores=NC, num_subcores=NS
    )


def _tile_chunks(gid):
    start = gid * BASE + lax.min(gid, XTRA)
    return start, gid < XTRA


def _stage_indices(e3, which, idx_v, gid):
    start, extra = _tile_chunks(gid)
    pltpu.sync_copy(e3.at[which, pl.ds(start, BASE)], idx_v.at[pl.ds(0, BASE)])

    @pl.when(extra)
    def _():
        pltpu.sync_copy(e3.at[which, pl.ds(start + BASE, 1)],
                        idx_v.at[pl.ds(BASE, 1)])


def _make_degree_kernel():
    @functools.partial(
        pl.kernel,
        out_type=(
            jax.ShapeDtypeStruct((NC, NP), jnp.float32),
            jax.ShapeDtypeStruct((NC, NP), jnp.float32),
        ),
        mesh=_sc_mesh(),
        compiler_params=_SC_PARAMS,
        scratch_types=[
            pltpu.VMEM((BASE + 1, CHUNK), jnp.int32),
            pltpu.VMEM((BASE + 1, CHUNK), jnp.int32),
            pltpu.VMEM((CHUNK,), jnp.float32),
            pltpu.VMEM_SHARED((NP,), jnp.float32),
            pltpu.VMEM_SHARED((NP,), jnp.float32),
            pltpu.SemaphoreType.DMA,
            pltpu.SemaphoreType.DMA,
        ],
    )
    def deg_kernel(e3_hbm, ones_hbm, zeros_hbm, dog_hbm, dig_hbm,
                   src_v, dst_v, ones_v, acc_o, acc_i, sem_o, sem_i):
        c = lax.axis_index("c")
        s = lax.axis_index("s")
        gid = c * NS + s
        _stage_indices(e3_hbm, 0, src_v, gid)
        _stage_indices(e3_hbm, 1, dst_v, gid)
        pltpu.sync_copy(ones_hbm, ones_v)
        sl = pl.ds(s * ROWS_PER_TILE, ROWS_PER_TILE)
        pltpu.sync_copy(zeros_hbm, acc_o.at[sl])
        pltpu.sync_copy(zeros_hbm, acc_i.at[sl])
        plsc.subcore_barrier()
        _, extra = _tile_chunks(gid)

        def body(j, carry):
            pltpu.async_copy(ones_v, acc_o.at[src_v.at[j]], sem_o, add=True)
            pltpu.async_copy(ones_v, acc_i.at[dst_v.at[j]], sem_i, add=True)
            return carry

        lax.fori_loop(0, BASE, body, 0)

        @pl.when(extra)
        def _():
            pltpu.async_copy(ones_v, acc_o.at[src_v.at[BASE]], sem_o, add=True)
            pltpu.async_copy(ones_v, acc_i.at[dst_v.at[BASE]], sem_i, add=True)

        def drain(j, carry):
            pltpu.make_async_copy(ones_v, acc_o.at[src_v.at[j]], sem_o).wait()
            pltpu.make_async_copy(ones_v, acc_i.at[dst_v.at[j]], sem_i).wait()
            return carry

        lax.fori_loop(0, BASE, drain, 0)

        @pl.when(extra)
        def _():
            pltpu.make_async_copy(ones_v, acc_o.at[src_v.at[BASE]],
                                  sem_o).wait()
            pltpu.make_async_copy(ones_v, acc_i.at[dst_v.at[BASE]],
                                  sem_i).wait()

        plsc.subcore_barrier()
        pltpu.sync_copy(acc_o.at[sl], dog_hbm.at[c, sl])
        pltpu.sync_copy(acc_i.at[sl], dig_hbm.at[c, sl])

    return deg_kernel


def _make_agg_kernel(nh, d):
    @functools.partial(
        pl.kernel,
        out_type=jax.ShapeDtypeStruct((NC, NP, 128), jnp.float32),
        mesh=_sc_mesh(),
        compiler_params=_SC_PARAMS,
        scratch_types=[
            pltpu.VMEM((BASE + 1, CHUNK), jnp.int32),
            pltpu.VMEM((BASE + 1, CHUNK), jnp.int32),
            [pltpu.VMEM((CHUNK, d), jnp.float32) for _ in range(NBUF)],
            pltpu.VMEM_SHARED((NP, d), jnp.float32),
            [pltpu.SemaphoreType.DMA for _ in range(NBUF)],
            [pltpu.SemaphoreType.DMA for _ in range(NBUF)],
        ],
    )
    def agg_kernel(*refs):
        feats = refs[:nh]
        e3_hbm, zeros_hbm, out_hbm = refs[nh:nh + 3]
        src_v, dst_v, rows, acc, gsem, ssem = refs[nh + 3:]
        c = lax.axis_index("c")
        s = lax.axis_index("s")
        gid = c * NS + s
        _stage_indices(e3_hbm, 0, src_v, gid)
        _stage_indices(e3_hbm, 1, dst_v, gid)
        _, extra = _tile_chunks(gid)
        sl = pl.ds(s * ROWS_PER_TILE, ROWS_PER_TILE)

        for h in range(nh):
            feat = feats[h]
            pltpu.sync_copy(zeros_hbm, acc.at[sl])
            plsc.subcore_barrier()

            @pl.when(extra)
            def _():
                pltpu.async_copy(feat.at[src_v.at[BASE]], rows[0], gsem[0])
                pltpu.make_async_copy(feat.at[src_v.at[BASE]], rows[0],
                                      gsem[0]).wait()
                pltpu.sync_copy(rows[0], acc.at[dst_v.at[BASE]], add=True)

            for b in range(NBUF):
                pltpu.async_copy(feat.at[src_v.at[b]], rows[b], gsem[b])

            def body(jj, carry):
                j = jj * NBUF
                for b in range(NBUF):
                    pltpu.make_async_copy(feat.at[src_v.at[j + b]], rows[b],
                                          gsem[b]).wait()
                    pltpu.async_copy(rows[b], acc.at[dst_v.at[j + b]],
                                     ssem[b], add=True)

                    @pl.when(j + b + NBUF < BASE)
                    def _():
                        pltpu.make_async_copy(rows[b],
                                              acc.at[dst_v.at[j + b]],
                                              ssem[b]).wait()
                        pltpu.async_copy(feat.at[src_v.at[j + b + NBUF]],
                                         rows[b], gsem[b])
                return carry

            lax.fori_loop(0, BASE // NBUF, body, 0)
            for b in range(NBUF):
                pltpu.make_async_copy(rows[b],
                                      acc.at[dst_v.at[BASE - NBUF + b]],
                                      ssem[b]).wait()
            plsc.subcore_barrier()
            pltpu.sync_copy(acc.at[sl],
                            out_hbm.at[c, sl, pl.ds(h * d, d)])

    return agg_kernel


def _tc_a_body(x_ref, w1_ref, dog_ref, dig_ref,
               xwa_ref, xwb_ref, normo_ref, normi_ref):
    do = dog_ref[0, :] + dog_ref[1, :]
    di = dig_ref[0, :] + dig_ref[1, :]
    no = jnp.where(do > 0, lax.rsqrt(do), 0.0)
    ni = jnp.where(di > 0, lax.rsqrt(di), 0.0)
    normo_ref[0, :] = no
    normi_ref[0, :] = ni
    xw = jnp.dot(x_ref[...], w1_ref[...], preferred_element_type=jnp.float32)
    xw = xw * no[:N, None]
    xwa_ref[...] = xw[:, :HW]
    xwb_ref[...] = xw[:, HW:]


def _tc_b_body(p_ref, normi_ref, normo_ref, b1_ref, w2_ref, hw_ref):
    ni = normi_ref[0, :N]
    no = normo_ref[0, :N]
    a = p_ref[0, :N, :] + p_ref[1, :N, :]
    h = jnp.maximum(a * ni[:, None] + b1_ref[0, :][None, :], 0.0)
    hw = jnp.dot(h, w2_ref[...], preferred_element_type=jnp.float32)
    hw_ref[...] = hw * no[:, None]


def _tc_c_body(p_ref, normi_ref, b2_ref, out_ref):
    ni = normi_ref[0, :N]
    z = (p_ref[0, :N, :CP] + p_ref[1, :N, :CP]) * ni[:, None] \
        + b2_ref[0, :][None, :]
    col = lax.broadcasted_iota(jnp.int32, z.shape, 1)
    valid = col < N_CLS
    zm = jnp.where(valid, z, -jnp.inf)
    m = jnp.max(zm, axis=1, keepdims=True)
    e = jnp.where(valid, jnp.exp(z - m), 0.0)
    ssum = jnp.sum(e, axis=1, keepdims=True)
    out_ref[...] = (z - m - jnp.log(ssum))[:, :N_CLS]


def kernel(inputs, edge_index, W1, b1, W2, b2):
    x = inputs.astype(jnp.float32)
    e3 = edge_index.astype(jnp.int32).reshape(2, NCHUNKS, CHUNK)

    w2_p = jnp.pad(W2.astype(jnp.float32), ((0, 0), (0, CP - N_CLS)))
    b1_r = b1.astype(jnp.float32).reshape(1, D_HID)
    b2_r = jnp.pad(b2.astype(jnp.float32), (0, CP - N_CLS)).reshape(1, CP)

    ones_c = jnp.ones((CHUNK,), jnp.float32)
    zeros_t = jnp.zeros((ROWS_PER_TILE,), jnp.float32)
    zeros_th = jnp.zeros((ROWS_PER_TILE, HW), jnp.float32)
    zeros_tc = jnp.zeros((ROWS_PER_TILE, CP), jnp.float32)

    dog, dig = _make_degree_kernel()(e3, ones_c, zeros_t)

    xwa, xwb, normo, normi = pl.pallas_call(
        _tc_a_body,
        out_shape=[
            jax.ShapeDtypeStruct((N, HW), jnp.float32),
            jax.ShapeDtypeStruct((N, HW), jnp.float32),
            jax.ShapeDtypeStruct((1, NP), jnp.float32),
            jax.ShapeDtypeStruct((1, NP), jnp.float32),
        ],
    )(x, W1.astype(jnp.float32), dog, dig)

    agg1 = _make_agg_kernel(2, HW)(xwa, xwb, e3, zeros_th)

    hw2 = pl.pallas_call(
        _tc_b_body,
        out_shape=jax.ShapeDtypeStruct((N, CP), jnp.float32),
    )(agg1, normi, normo, b1_r, w2_p)

    agg2 = _make_agg_kernel(1, CP)(hw2, e3, zeros_tc)

    out = pl.pallas_call(
        _tc_c_body,
        out_shape=jax.ShapeDtypeStruct((N, N_CLS), jnp.float32),
    )(agg2, normi, b2_r)

    return out
